# baseline (speedup 1.0000x reference)
import jax
import jax.numpy as jnp
from jax import lax
from jax.experimental import pallas as pl
from jax.experimental.pallas import tpu as pltpu
from jax.experimental.pallas import tpu_sc as plsc

NUM_NODES = 1000000
DIM = 32
TOTAL = NUM_NODES * DIM
WIDE = 128
ROWS = TOTAL // WIDE
NUM_CORES = 2
ROWS_PER_CORE = ROWS // NUM_CORES

PIECE = 5000
NPIECES = ROWS_PER_CORE // PIECE
NBUF = 2


def _copy_body(w_hbm, out_hbm, b0, b1, si0, si1, so0, so1):
    bufs = (b0, b1)
    isems = (si0, si1)
    osems = (so0, so1)
    base = lax.axis_index("c") * ROWS_PER_CORE

    def in_cp(p, b):
        return pltpu.make_async_copy(
            w_hbm.at[pl.ds(base + p * PIECE, PIECE)], bufs[b], isems[b])

    def out_cp(p, b):
        return pltpu.make_async_copy(
            bufs[b], out_hbm.at[pl.ds(base + p * PIECE, PIECE)], osems[b])

    in_cp(0, 0).start()
    for p in range(NPIECES):
        b = p % NBUF
        in_cp(p, b).wait()
        out_cp(p, b).start()
        if p + 1 < NPIECES:
            nb = (p + 1) % NBUF
            if p - 1 >= 0:
                out_cp(p - 1, nb).wait()
            in_cp(p + 1, nb).start()
    out_cp(NPIECES - 2, (NPIECES - 2) % NBUF).wait()
    out_cp(NPIECES - 1, (NPIECES - 1) % NBUF).wait()


def kernel(W):
    mesh = plsc.ScalarSubcoreMesh(axis_name="c")
    flat = W.reshape(ROWS, WIDE)
    out = pl.kernel(
        _copy_body,
        out_type=jax.ShapeDtypeStruct((ROWS, WIDE), jnp.float32),
        mesh=mesh,
        scratch_types=(
            [pltpu.MemorySpace.VMEM_SHARED((PIECE, WIDE), jnp.float32)
             for _ in range(NBUF)]
            + [pltpu.SemaphoreType.DMA for _ in range(2 * NBUF)]
        ),
    )(flat)
    return out.reshape(NUM_NODES, DIM)
